# baseline (speedup 1.0000x reference)
import functools

import jax
import jax.numpy as jnp
from jax import lax
from jax.experimental import pallas as pl
from jax.experimental.pallas import tpu as pltpu
from jax.experimental.pallas import tpu_sc as plsc

N_EMB = 1000000
D_EMB = 32
BATCH = 16384

L = 16
NC = 2
NS = 16
NW = NC * NS
B_PER_W = BATCH // NW
CHUNK = 64
N_CHUNKS = B_PER_W // CHUNK
C_GROUPS = CHUNK // L


@functools.partial(
    pl.kernel,
    mesh=plsc.VectorSubcoreMesh(core_axis_name="c", subcore_axis_name="s"),
    out_type=jax.ShapeDtypeStruct((BATCH,), jnp.float32),
    scratch_types=[
        pltpu.VMEM((B_PER_W,), jnp.int32),
        pltpu.VMEM((B_PER_W,), jnp.int32),
        pltpu.VMEM((CHUNK, 8, D_EMB), jnp.float32),
        pltpu.VMEM(((D_EMB + 1) * L,), jnp.float32),
        pltpu.VMEM((B_PER_W,), jnp.float32),
        pltpu.SemaphoreType.DMA,
    ],
    compiler_params=pltpu.CompilerParams(needs_layout_passes=False),
)
def _sc_gather_dot(idx_hbm, table3_hbm, wb_hbm, out_hbm,
                   idx_v, tid_v, slabs_v, wb_v, out_v, sem):
    wid = lax.axis_index("s") * NC + lax.axis_index("c")
    base = wid * B_PER_W

    pltpu.sync_copy(idx_hbm.at[pl.ds(base, B_PER_W)], idx_v)
    pltpu.sync_copy(wb_hbm, wb_v)

    def tid_body(t, carry):
        t0 = t * L
        tid_v[pl.ds(t0, L)] = lax.shift_right_logical(idx_v[pl.ds(t0, L)], 3)
        return carry

    lax.fori_loop(0, B_PER_W // L, tid_body, 0)

    ws = [wb_v[pl.ds(d * L, L)] for d in range(D_EMB)]
    bias = wb_v[pl.ds(D_EMB * L, L)]
    lane = lax.iota(jnp.int32, L)

    def chunk_body(k, carry):
        k0 = k * CHUNK
        copies = []
        for t in range(CHUNK // L):
            tv = tid_v[pl.ds(k0 + t * L, L)]
            for l in range(L):
                copies.append(pltpu.async_copy(
                    table3_hbm.at[tv[l]], slabs_v.at[t * L + l], sem))
        for cp in copies:
            cp.wait()
        for g in range(C_GROUPS):
            row0 = k0 + g * L
            idx16 = idx_v[pl.ds(row0, L)]
            sub = lax.bitwise_and(idx16, 7)
            cvec = lane + g * L
            acc = bias
            for d in range(D_EMB):
                col = plsc.load_gather(
                    slabs_v,
                    [cvec, sub, jnp.full((L,), d, dtype=jnp.int32)])
                acc = acc + col * ws[d]
            out_v[pl.ds(row0, L)] = acc
        return carry

    lax.fori_loop(0, N_CHUNKS, chunk_body, 0)

    pltpu.sync_copy(out_v, out_hbm.at[pl.ds(base, B_PER_W)])


def kernel(idx, table, W, b):
    table3 = table.reshape(N_EMB // 8, 8, D_EMB)
    wb = jnp.concatenate(
        [
            jnp.broadcast_to(W.reshape(D_EMB, 1), (D_EMB, L)),
            jnp.broadcast_to(b.reshape(1, 1), (1, L)),
        ],
        axis=0,
    ).reshape((D_EMB + 1) * L)
    y = _sc_gather_dot(idx.astype(jnp.int32), table3, wb)
    return y.reshape(BATCH, 1)


# speedup vs baseline: 1.0228x; 1.0228x over previous
import functools

import jax
import jax.numpy as jnp
from jax import lax
from jax.experimental import pallas as pl
from jax.experimental.pallas import tpu as pltpu
from jax.experimental.pallas import tpu_sc as plsc

N_EMB = 1000000
D_EMB = 32
BATCH = 16384

L = 16
NC = 2
NS = 16
NW = NC * NS
B_PER_W = BATCH // NW
CHUNK = 32
N_CHUNKS = B_PER_W // CHUNK
C_GROUPS = CHUNK // L


@functools.partial(
    pl.kernel,
    mesh=plsc.VectorSubcoreMesh(core_axis_name="c", subcore_axis_name="s"),
    out_type=jax.ShapeDtypeStruct((BATCH,), jnp.float32),
    scratch_types=[
        pltpu.VMEM((B_PER_W,), jnp.int32),
        pltpu.VMEM((B_PER_W,), jnp.int32),
        pltpu.VMEM((2, CHUNK, 8, D_EMB), jnp.float32),
        pltpu.VMEM(((D_EMB + 1) * L,), jnp.float32),
        pltpu.VMEM((B_PER_W,), jnp.float32),
        [pltpu.SemaphoreType.DMA] * 2,
    ],
    compiler_params=pltpu.CompilerParams(needs_layout_passes=False),
)
def _sc_gather_dot(idx_hbm, table3_hbm, wb_hbm, out_hbm,
                   idx_v, tid_v, slabs_v, wb_v, out_v, sems):
    wid = lax.axis_index("s") * NC + lax.axis_index("c")
    base = wid * B_PER_W

    pltpu.sync_copy(idx_hbm.at[pl.ds(base, B_PER_W)], idx_v)
    pltpu.sync_copy(wb_hbm, wb_v)

    def tid_body(t, carry):
        t0 = t * L
        tid_v[pl.ds(t0, L)] = lax.shift_right_logical(idx_v[pl.ds(t0, L)], 3)
        return carry

    lax.fori_loop(0, B_PER_W // L, tid_body, 0)

    ws = [wb_v[pl.ds(d * L, L)] for d in range(D_EMB)]
    bias = wb_v[pl.ds(D_EMB * L, L)]
    lane = lax.iota(jnp.int32, L)

    def fire(k, b):
        k0 = k * CHUNK
        for t in range(CHUNK // L):
            tv = tid_v[pl.ds(k0 + t * L, L)]
            for l in range(L):
                pltpu.async_copy(
                    table3_hbm.at[tv[l]], slabs_v.at[b, t * L + l], sems[b])

    def drain(b):
        pltpu.make_async_copy(
            table3_hbm.at[pl.ds(0, CHUNK)], slabs_v.at[b], sems[b]).wait()

    def compute(k, b):
        k0 = k * CHUNK
        bsplat = jnp.full((L,), b, dtype=jnp.int32)
        for g in range(C_GROUPS):
            row0 = k0 + g * L
            idx16 = idx_v[pl.ds(row0, L)]
            sub = lax.bitwise_and(idx16, 7)
            cvec = lane + g * L
            acc = bias
            for d in range(D_EMB):
                col = plsc.load_gather(
                    slabs_v,
                    [bsplat, cvec, sub, jnp.full((L,), d, dtype=jnp.int32)])
                acc = acc + col * ws[d]
            out_v[pl.ds(row0, L)] = acc

    fire(0, 0)

    def pair_body(kk, carry):
        for b in range(2):
            k = kk * 2 + b

            @pl.when(k + 1 < N_CHUNKS)
            def _():
                fire(k + 1, 1 - b)

            drain(b)
            compute(k, b)
        return carry

    lax.fori_loop(0, N_CHUNKS // 2, pair_body, 0)

    pltpu.sync_copy(out_v, out_hbm.at[pl.ds(base, B_PER_W)])


def kernel(idx, table, W, b):
    table3 = table.reshape(N_EMB // 8, 8, D_EMB)
    wb = jnp.concatenate(
        [
            jnp.broadcast_to(W.reshape(D_EMB, 1), (D_EMB, L)),
            jnp.broadcast_to(b.reshape(1, 1), (1, L)),
        ],
        axis=0,
    ).reshape((D_EMB + 1) * L)
    y = _sc_gather_dot(idx.astype(jnp.int32), table3, wb)
    return y.reshape(BATCH, 1)
